# baseline (speedup 1.0000x reference)
import functools

import jax
import jax.numpy as jnp
from jax import lax
from jax.experimental import pallas as pl
from jax.experimental.pallas import tpu as pltpu
from jax.experimental.pallas import tpu_sc as plsc

_VOCAB = 30522
_EMB_DIM = 768
_NUM_CLASSES = 2
_BATCH = 4096
_SEQ = 50

_NC = 2
_NS = 16
_L = 16
_NW = _NC * _NS
_B_PER_W = _BATCH // _NW
_G = _B_PER_W // _L

_ROW_BLK = 4096
_SPLIT = 16384
_V2 = _VOCAB - _SPLIT


def _proj_body(e_ref, w_ref, b_ref, o_ref):
    o_ref[...] = (
        lax.dot_general(
            e_ref[...], w_ref[...],
            (((1,), (1,)), ((), ())),
            preferred_element_type=jnp.float32,
        )
        + b_ref[...]
    )


def _project_rows(emb_table, W, b, blk_start, n_rows):
    n_blks = (n_rows + _ROW_BLK - 1) // _ROW_BLK
    return pl.pallas_call(
        _proj_body,
        grid=(n_blks,),
        in_specs=[
            pl.BlockSpec((_ROW_BLK, _EMB_DIM), lambda i: (i + blk_start, 0)),
            pl.BlockSpec((_NUM_CLASSES, _EMB_DIM), lambda i: (0, 0)),
            pl.BlockSpec((1, _NUM_CLASSES), lambda i: (0, 0)),
        ],
        out_specs=pl.BlockSpec((_ROW_BLK, _NUM_CLASSES), lambda i: (i, 0)),
        out_shape=jax.ShapeDtypeStruct((n_rows, _NUM_CLASSES), jnp.float32),
    )(emb_table, W, b.reshape(1, _NUM_CLASSES))


def _make_pool(chunk_start, chunk_rows, is_first):
    mesh = plsc.VectorSubcoreMesh(core_axis_name="c", subcore_axis_name="s")
    scratch = [
        pltpu.VMEM((chunk_rows * _NUM_CLASSES,), jnp.float32),
        pltpu.VMEM((_B_PER_W * _SEQ,), jnp.int32),
        pltpu.VMEM((_B_PER_W * _NUM_CLASSES,), jnp.float32),
    ]

    def body(p_hbm, ids_hbm, *rest):
        if is_first:
            o_hbm, p_v, ids_v, o_v = rest
        else:
            part_hbm, o_hbm, p_v, ids_v, o_v = rest
        wid = lax.axis_index("s") * _NC + lax.axis_index("c")
        base = wid * _B_PER_W * _NUM_CLASSES

        pltpu.sync_copy(p_hbm, p_v)
        pltpu.sync_copy(
            ids_hbm.at[pl.ds(wid * _B_PER_W * _SEQ, _B_PER_W * _SEQ)], ids_v
        )
        if not is_first:
            pltpu.sync_copy(
                part_hbm.at[pl.ds(base, _B_PER_W * _NUM_CLASSES)], o_v
            )

        inv_s = jnp.float32(1.0 / _SEQ)
        lanes = lax.iota(jnp.int32, _L)
        lo = jnp.int32(chunk_start)
        hi = jnp.int32(chunk_start + chunk_rows)
        zerof = jnp.zeros((_L,), jnp.float32)

        def step(t, accs):
            out = []
            for g in range(_G):
                a0, a1 = accs[g]
                idx = ids_v[
                    pl.ds(pl.multiple_of(t * (_G * _L) + g * _L, _L), _L)
                ]
                valid = (idx >= lo) & (idx < hi)
                rel = jnp.minimum(
                    jnp.maximum(idx - lo, jnp.int32(0)),
                    jnp.int32(chunk_rows - 1),
                )
                rel2 = rel + rel
                g0 = plsc.load_gather(p_v, [rel2])
                g1 = plsc.load_gather(p_v, [rel2 + 1])
                a0 = a0 + jnp.where(valid, g0, zerof)
                a1 = a1 + jnp.where(valid, g1, zerof)
                out.append((a0, a1))
            return tuple(out)

        accs = lax.fori_loop(
            0, _SEQ, step, tuple((zerof, zerof) for _ in range(_G))
        )

        for g in range(_G):
            a0, a1 = accs[g]
            pos = (lanes + g * _L) * _NUM_CLASSES
            if is_first:
                plsc.store_scatter(o_v, [pos], a0 * inv_s)
                plsc.store_scatter(o_v, [pos + 1], a1 * inv_s)
            else:
                plsc.addupdate_scatter(o_v, [pos], a0 * inv_s)
                plsc.addupdate_scatter(o_v, [pos + 1], a1 * inv_s)

        pltpu.sync_copy(o_v, o_hbm.at[pl.ds(base, _B_PER_W * _NUM_CLASSES)])

    return pl.kernel(
        body,
        mesh=mesh,
        compiler_params=pltpu.CompilerParams(needs_layout_passes=False),
        out_type=jax.ShapeDtypeStruct((_BATCH * _NUM_CLASSES,), jnp.float32),
        scratch_types=scratch,
    )


_pool_a = _make_pool(0, _SPLIT, True)
_pool_b = _make_pool(_SPLIT, _V2, False)


def kernel(input_ids, emb_table, W, b):
    ids = (
        input_ids.astype(jnp.int32)
        .reshape(_NW, _B_PER_W, _SEQ)
        .transpose(0, 2, 1)
        .reshape(-1)
    )
    p1 = _project_rows(emb_table, W, b, 0, _SPLIT).reshape(-1)
    p2 = _project_rows(emb_table, W, b, _SPLIT // _ROW_BLK, _V2).reshape(-1)
    partial = _pool_a(p1, ids)
    out = _pool_b(p2, ids, partial)
    logits = out.reshape(_BATCH, _NUM_CLASSES)
    return (logits, logits)


# speedup vs baseline: 1.1373x; 1.1373x over previous
import functools

import jax
import jax.numpy as jnp
from jax import lax
from jax.experimental import pallas as pl
from jax.experimental.pallas import tpu as pltpu
from jax.experimental.pallas import tpu_sc as plsc

_VOCAB = 30522
_EMB_DIM = 768
_NUM_CLASSES = 2
_BATCH = 4096
_SEQ = 50

_NC = 2
_NS = 16
_L = 16
_NW = _NC * _NS
_B_PER_W = _BATCH // _NW
_G = _B_PER_W // _L

_ROW_BLK = 4096
_N_BLKS = (_VOCAB + _ROW_BLK - 1) // _ROW_BLK
_VOCAB_PAD = 30528


def _proj_body(e_ref, w_ref, b_ref, o_ref):
    o_ref[...] = (
        lax.dot_general(
            e_ref[...], w_ref[...],
            (((1,), (1,)), ((), ())),
            preferred_element_type=jnp.float32,
        )
        + b_ref[...]
    )


def _project_table(emb_table, W, b):
    return pl.pallas_call(
        _proj_body,
        grid=(_N_BLKS,),
        in_specs=[
            pl.BlockSpec((_ROW_BLK, _EMB_DIM), lambda i: (i, 0)),
            pl.BlockSpec((_NUM_CLASSES, _EMB_DIM), lambda i: (0, 0)),
            pl.BlockSpec((1, _NUM_CLASSES), lambda i: (0, 0)),
        ],
        out_specs=pl.BlockSpec((_ROW_BLK, _NUM_CLASSES), lambda i: (i, 0)),
        out_shape=jax.ShapeDtypeStruct((_VOCAB_PAD, _NUM_CLASSES), jnp.float32),
        compiler_params=pltpu.CompilerParams(
            dimension_semantics=("parallel",),
        ),
    )(emb_table, W, b.reshape(1, _NUM_CLASSES))


@functools.partial(
    pl.kernel,
    mesh=plsc.VectorSubcoreMesh(core_axis_name="c", subcore_axis_name="s"),
    compiler_params=pltpu.CompilerParams(needs_layout_passes=False),
    out_type=jax.ShapeDtypeStruct((_BATCH * _NUM_CLASSES,), jnp.float32),
    scratch_types=[
        pltpu.VMEM((_VOCAB_PAD * _NUM_CLASSES,), jnp.float32),
        pltpu.VMEM((_B_PER_W * _SEQ,), jnp.int32),
        pltpu.VMEM((_B_PER_W * _NUM_CLASSES,), jnp.float32),
    ],
)
def _pool_kernel(p_hbm, ids_hbm, o_hbm, p_v, ids_v, o_v):
    wid = lax.axis_index("s") * _NC + lax.axis_index("c")
    q_len = (_VOCAB_PAD * _NUM_CLASSES) // 4
    for q in range(4):
        off = lax.rem(wid + q, 4) * q_len
        pltpu.sync_copy(
            p_hbm.at[pl.ds(off, q_len)], p_v.at[pl.ds(off, q_len)]
        )
    pltpu.sync_copy(ids_hbm.at[pl.ds(wid * _B_PER_W * _SEQ, _B_PER_W * _SEQ)], ids_v)

    inv_s = jnp.float32(1.0 / _SEQ)
    lanes = lax.iota(jnp.int32, _L)

    def body(t, accs):
        out = []
        for g in range(_G):
            a0, a1 = accs[g]
            idx = ids_v[pl.ds(pl.multiple_of(t * (_G * _L) + g * _L, _L), _L)]
            idx2 = idx + idx
            a0 = a0 + plsc.load_gather(p_v, [idx2])
            a1 = a1 + plsc.load_gather(p_v, [idx2 + 1])
            out.append((a0, a1))
        return tuple(out)

    zero = jnp.zeros((_L,), jnp.float32)
    accs = lax.fori_loop(0, _SEQ, body, tuple((zero, zero) for _ in range(_G)))

    for g in range(_G):
        a0, a1 = accs[g]
        pos = (lanes + g * _L) * _NUM_CLASSES
        plsc.store_scatter(o_v, [pos], a0 * inv_s)
        plsc.store_scatter(o_v, [pos + 1], a1 * inv_s)

    base = wid * _B_PER_W * _NUM_CLASSES
    pltpu.sync_copy(o_v, o_hbm.at[pl.ds(base, _B_PER_W * _NUM_CLASSES)])


def kernel(input_ids, emb_table, W, b):
    ids = (
        input_ids.astype(jnp.int32)
        .reshape(_NW, _B_PER_W, _SEQ)
        .transpose(0, 2, 1)
        .reshape(-1)
    )
    p = _project_table(emb_table, W, b).reshape(-1)
    out = _pool_kernel(p, ids)
    logits = out.reshape(_BATCH, _NUM_CLASSES)
    return (logits, logits)
